# Initial kernel scaffold; baseline (speedup 1.0000x reference)
import jax
import jax.numpy as jnp
from jax.experimental import pallas as pl


def kernel(_input, values, bias, rows, cols):
    raise NotImplementedError("write your pallas kernel here")


# speedup vs baseline: 4.4565x; 4.4565x over previous
import functools

import jax
import jax.numpy as jnp
from jax import lax
from jax.experimental import pallas as pl
from jax.experimental.pallas import tpu as pltpu
from jax.experimental.pallas import tpu_sc as plsc

IN_F = 16384
OUT_F = 16384
B = 128
NW = 32
RPW = OUT_F // NW
K = 128
NST = 48


def _spmm_body(xT, vals, rows, cols, bias, starts, out,
               accv, chunkv, idxv, valv, rlocv, biasv, startv, sem):
    cid = lax.axis_index("c")
    sid = lax.axis_index("s")
    wid = cid * 16 + sid
    base = wid * RPW

    pltpu.sync_copy(starts, startv)
    se = startv[pl.ds(wid, 16)]
    lo = se[0]
    hi = se[1]

    pltpu.sync_copy(bias.at[pl.ds(base, RPW)], biasv)

    def init_rows(g, carry):
        bv16 = biasv[pl.ds(g * 16, 16)]
        for l in range(16):
            bv = jnp.full((16,), bv16[l], jnp.float32)
            for j in range(B // 16):
                accv[g * 16 + l, pl.ds(j * 16, 16)] = bv
        return carry

    lax.fori_loop(0, RPW // 16, init_rows, 0)

    c0 = lo // K
    c1 = lax.div(hi + (K - 1), K)

    def chunk_body(ci, carry):
        off = ci * K
        pltpu.sync_copy(cols.at[pl.ds(off, K)], idxv)
        cp = pltpu.async_copy(xT.at[idxv], chunkv, sem)
        pltpu.sync_copy(rows.at[pl.ds(off, K)], rlocv)
        pltpu.sync_copy(vals.at[pl.ds(off, K)], valv)
        for g in range(K // 16):
            gi = off + g * 16 + lax.iota(jnp.int32, 16)
            m = (gi >= lo) & (gi < hi)
            valv[pl.ds(g * 16, 16)] = jnp.where(
                m, valv[pl.ds(g * 16, 16)], jnp.float32(0.0))
            r = rlocv[pl.ds(g * 16, 16)] - base
            rlocv[pl.ds(g * 16, 16)] = jnp.clip(r, 0, RPW - 1)
        cp.wait()

        def acc_body(g, carry2):
            r16 = rlocv[pl.ds(g * 16, 16)]
            v16 = valv[pl.ds(g * 16, 16)]
            for l in range(16):
                r = r16[l]
                v = v16[l]
                e = g * 16 + l
                for j in range(B // 16):
                    plsc.addupdate(accv.at[r, pl.ds(j * 16, 16)],
                                   v * chunkv[e, pl.ds(j * 16, 16)])
            return carry2

        lax.fori_loop(0, K // 16, acc_body, 0)
        return carry

    lax.fori_loop(c0, c1, chunk_body, 0)

    pltpu.sync_copy(accv, out.at[pl.ds(base, RPW)])


@jax.jit
def _spmm(xT, vals, rows, cols, bias, starts):
    mesh = plsc.VectorSubcoreMesh(core_axis_name="c", subcore_axis_name="s")
    run = functools.partial(
        pl.kernel,
        mesh=mesh,
        out_type=jax.ShapeDtypeStruct((OUT_F, B), jnp.float32),
        scratch_types=[
            pltpu.VMEM((RPW, B), jnp.float32),
            pltpu.VMEM((K, B), jnp.float32),
            pltpu.VMEM((K,), jnp.int32),
            pltpu.VMEM((K,), jnp.float32),
            pltpu.VMEM((K,), jnp.int32),
            pltpu.VMEM((RPW,), jnp.float32),
            pltpu.VMEM((NST,), jnp.int32),
            pltpu.SemaphoreType.DMA,
        ],
    )(_spmm_body)
    return run(xT, vals, rows, cols, bias, starts)


def kernel(_input, values, bias, rows, cols):
    rows32 = rows.astype(jnp.int32)
    cols32 = cols.astype(jnp.int32)
    nnz = rows32.shape[0]
    pad = (-nnz) % K
    rows_p = jnp.pad(rows32, (0, pad))
    cols_p = jnp.pad(cols32, (0, pad))
    vals_p = jnp.pad(values, (0, pad))
    bounds = jnp.arange(NW + 1, dtype=jnp.int32) * RPW
    starts = jnp.searchsorted(rows32, bounds).astype(jnp.int32)
    starts = jnp.pad(starts, (0, NST - (NW + 1)))
    xT = _input.T.copy()
    out_t = _spmm(xT, vals_p, rows_p, cols_p, bias, starts)
    return out_t.T


# speedup vs baseline: 8.6308x; 1.9367x over previous
import functools

import jax
import jax.numpy as jnp
from jax import lax
from jax.experimental import pallas as pl
from jax.experimental.pallas import tpu as pltpu
from jax.experimental.pallas import tpu_sc as plsc

IN_F = 16384
OUT_F = 16384
B = 128
NW = 32
RPW = OUT_F // NW
K = 128
NST = 48


def _spmm_body(xT, vals, rows, cols, bias, starts, out,
               accv, chunkv, idxv, valv, rlocv, biasv, startv, sem):
    cid = lax.axis_index("c")
    sid = lax.axis_index("s")
    wid = cid * 16 + sid
    base = wid * RPW

    pltpu.sync_copy(starts, startv)
    se = startv[pl.ds(wid, 16)]
    lo = se[0]
    hi = se[1]

    pltpu.sync_copy(bias.at[pl.ds(base, RPW)], biasv)

    def init_rows(g, carry):
        bv16 = biasv[pl.ds(g * 16, 16)]
        for l in range(16):
            bv = jnp.full((16,), bv16[l], jnp.float32)
            for j in range(B // 16):
                accv[g * 16 + l, pl.ds(j * 16, 16)] = bv
        return carry

    lax.fori_loop(0, RPW // 16, init_rows, 0)

    c0 = lo // K
    c1 = lax.div(hi + (K - 1), K)

    def chunk_body(ci, carry):
        off = ci * K
        pltpu.sync_copy(cols.at[pl.ds(off, K)], idxv)
        cp = pltpu.async_copy(xT.at[idxv], chunkv, sem)
        pltpu.sync_copy(rows.at[pl.ds(off, K)], rlocv)
        pltpu.sync_copy(vals.at[pl.ds(off, K)], valv)
        for g in range(K // 16):
            gi = off + g * 16 + lax.iota(jnp.int32, 16)
            m = (gi >= lo) & (gi < hi)
            valv[pl.ds(g * 16, 16)] = jnp.where(
                m, valv[pl.ds(g * 16, 16)], jnp.float32(0.0))
            r = rlocv[pl.ds(g * 16, 16)] - base
            rlocv[pl.ds(g * 16, 16)] = jnp.clip(r, 0, RPW - 1)
        cp.wait()

        NJ = B // 16

        def acc_body(g, carry2):
            r16 = rlocv[pl.ds(g * 16, 16)]
            v16 = valv[pl.ds(g * 16, 16)]

            def stage(l):
                v = v16[l]
                e = g * 16 + l
                return [v * chunkv[e, pl.ds(j * 16, 16)] for j in range(NJ)]

            prods = stage(0)
            for l in range(16):
                nxt = stage(l + 1) if l + 1 < 16 else None
                r = r16[l]
                for j in range(NJ):
                    plsc.addupdate(accv.at[r, pl.ds(j * 16, 16)], prods[j])
                prods = nxt
            return carry2

        lax.fori_loop(0, K // 16, acc_body, 0)
        return carry

    lax.fori_loop(c0, c1, chunk_body, 0)

    pltpu.sync_copy(accv, out.at[pl.ds(base, RPW)])


@jax.jit
def _spmm(xT, vals, rows, cols, bias, starts):
    mesh = plsc.VectorSubcoreMesh(core_axis_name="c", subcore_axis_name="s")
    run = functools.partial(
        pl.kernel,
        mesh=mesh,
        out_type=jax.ShapeDtypeStruct((OUT_F, B), jnp.float32),
        scratch_types=[
            pltpu.VMEM((RPW, B), jnp.float32),
            pltpu.VMEM((K, B), jnp.float32),
            pltpu.VMEM((K,), jnp.int32),
            pltpu.VMEM((K,), jnp.float32),
            pltpu.VMEM((K,), jnp.int32),
            pltpu.VMEM((RPW,), jnp.float32),
            pltpu.VMEM((NST,), jnp.int32),
            pltpu.SemaphoreType.DMA,
        ],
    )(_spmm_body)
    return run(xT, vals, rows, cols, bias, starts)


def kernel(_input, values, bias, rows, cols):
    rows32 = rows.astype(jnp.int32)
    cols32 = cols.astype(jnp.int32)
    nnz = rows32.shape[0]
    pad = (-nnz) % K
    rows_p = jnp.pad(rows32, (0, pad))
    cols_p = jnp.pad(cols32, (0, pad))
    vals_p = jnp.pad(values, (0, pad))
    bounds = jnp.arange(NW + 1, dtype=jnp.int32) * RPW
    starts = jnp.searchsorted(rows32, bounds).astype(jnp.int32)
    starts = jnp.pad(starts, (0, NST - (NW + 1)))
    xT = _input.T.copy()
    out_t = _spmm(xT, vals_p, rows_p, cols_p, bias, starts)
    return out_t.T


# speedup vs baseline: 12.7846x; 1.4813x over previous
import functools

import jax
import jax.numpy as jnp
from jax import lax
from jax.experimental import pallas as pl
from jax.experimental.pallas import tpu as pltpu
from jax.experimental.pallas import tpu_sc as plsc

IN_F = 16384
OUT_F = 16384
B = 128
NW = 32
RPW = OUT_F // NW
K = 128
NST = 48


def _spmm_body(xT, vals, rows, cols, bias, starts, out,
               accv, chunk0, chunk1, idx0, idx1, val0, val1, rloc0, rloc1,
               biasv, startv, sem0, sem1):
    chunkb = (chunk0, chunk1)
    idxb = (idx0, idx1)
    valb = (val0, val1)
    rlocb = (rloc0, rloc1)
    semb = (sem0, sem1)

    cid = lax.axis_index("c")
    sid = lax.axis_index("s")
    wid = cid * 16 + sid
    base = wid * RPW

    pltpu.sync_copy(starts, startv)
    se = startv[pl.ds(wid, 16)]
    lo = se[0]
    hi = se[1]

    pltpu.sync_copy(bias.at[pl.ds(base, RPW)], biasv)

    def init_rows(g, carry):
        bv16 = biasv[pl.ds(g * 16, 16)]
        for l in range(16):
            bv = jnp.full((16,), bv16[l], jnp.float32)
            for j in range(B // 16):
                accv[g * 16 + l, pl.ds(j * 16, 16)] = bv
        return carry

    lax.fori_loop(0, RPW // 16, init_rows, 0)

    c0 = lo // K
    c1 = lax.div(hi + (K - 1), K)

    def issue(b, ci):
        off = ci * K
        pltpu.sync_copy(cols.at[pl.ds(off, K)], idxb[b])
        pltpu.async_copy(xT.at[idxb[b]], chunkb[b], semb[b])
        pltpu.async_copy(rows.at[pl.ds(off, K)], rlocb[b], semb[b])
        pltpu.async_copy(vals.at[pl.ds(off, K)], valb[b], semb[b])

    def drain(b):
        pltpu.make_async_copy(xT.at[idxb[b]], chunkb[b], semb[b]).wait()
        pltpu.make_async_copy(rows.at[pl.ds(0, K)], rlocb[b], semb[b]).wait()
        pltpu.make_async_copy(vals.at[pl.ds(0, K)], valb[b], semb[b]).wait()

    def compute(b, ci):
        off = ci * K
        drain(b)
        for g in range(K // 16):
            gi = off + g * 16 + lax.iota(jnp.int32, 16)
            m = (gi >= lo) & (gi < hi)
            valb[b][pl.ds(g * 16, 16)] = jnp.where(
                m, valb[b][pl.ds(g * 16, 16)], jnp.float32(0.0))
            r = rlocb[b][pl.ds(g * 16, 16)] - base
            rlocb[b][pl.ds(g * 16, 16)] = jnp.clip(r, 0, RPW - 1)

        NJ = B // 16

        @plsc.parallel_loop(0, K // 16)
        def acc_body(g):
            r16 = rlocb[b][pl.ds(g * 16, 16)]
            v16 = valb[b][pl.ds(g * 16, 16)]

            def stage(l):
                v = v16[l]
                e = g * 16 + l
                return [v * chunkb[b][e, pl.ds(j * 16, 16)]
                        for j in range(NJ)]

            prods = stage(0)
            for l in range(16):
                nxt = stage(l + 1) if l + 1 < 16 else None
                r = r16[l]
                for j in range(NJ):
                    plsc.addupdate(accv.at[r, pl.ds(j * 16, 16)], prods[j])
                prods = nxt

    @pl.when(c0 < c1)
    def _():
        issue(0, c0)

    @pl.when(c0 + 1 < c1)
    def _():
        issue(1, c0 + 1)

    npairs = lax.div(c1 - c0 + 1, 2)

    def pair_body(t, carry):
        for b in range(2):
            ci = c0 + t * 2 + b

            @pl.when(ci < c1)
            def _():
                compute(b, ci)

                @pl.when(ci + 2 < c1)
                def _():
                    issue(b, ci + 2)

        return carry

    lax.fori_loop(0, npairs, pair_body, 0)

    pltpu.sync_copy(accv, out.at[pl.ds(base, RPW)])


@jax.jit
def _spmm(xT, vals, rows, cols, bias, starts):
    mesh = plsc.VectorSubcoreMesh(core_axis_name="c", subcore_axis_name="s")
    run = functools.partial(
        pl.kernel,
        mesh=mesh,
        out_type=jax.ShapeDtypeStruct((OUT_F, B), jnp.float32),
        scratch_types=[
            pltpu.VMEM((RPW, B), jnp.float32),
            pltpu.VMEM((K, B), jnp.float32),
            pltpu.VMEM((K, B), jnp.float32),
            pltpu.VMEM((K,), jnp.int32),
            pltpu.VMEM((K,), jnp.int32),
            pltpu.VMEM((K,), jnp.float32),
            pltpu.VMEM((K,), jnp.float32),
            pltpu.VMEM((K,), jnp.int32),
            pltpu.VMEM((K,), jnp.int32),
            pltpu.VMEM((RPW,), jnp.float32),
            pltpu.VMEM((NST,), jnp.int32),
            pltpu.SemaphoreType.DMA,
            pltpu.SemaphoreType.DMA,
        ],
    )(_spmm_body)
    return run(xT, vals, rows, cols, bias, starts)


def kernel(_input, values, bias, rows, cols):
    rows32 = rows.astype(jnp.int32)
    cols32 = cols.astype(jnp.int32)
    nnz = rows32.shape[0]
    pad = (-nnz) % K
    rows_p = jnp.pad(rows32, (0, pad))
    cols_p = jnp.pad(cols32, (0, pad))
    vals_p = jnp.pad(values, (0, pad))
    bounds = jnp.arange(NW + 1, dtype=jnp.int32) * RPW
    starts = jnp.searchsorted(rows32, bounds).astype(jnp.int32)
    starts = jnp.pad(starts, (0, NST - (NW + 1)))
    xT = _input.T.copy()
    out_t = _spmm(xT, vals_p, rows_p, cols_p, bias, starts)
    return out_t.T


# speedup vs baseline: 15.1086x; 1.1818x over previous
import functools

import jax
import jax.numpy as jnp
from jax import lax
from jax.experimental import pallas as pl
from jax.experimental.pallas import tpu as pltpu
from jax.experimental.pallas import tpu_sc as plsc

IN_F = 16384
OUT_F = 16384
B = 128
NW = 32
RPW = OUT_F // NW
K = 128
NST = 48


def _spmm_body(xT, vals, rows, cols, bias, starts, out,
               accv, chunk0, chunk1, idx0, idx1, val0, val1, rloc0, rloc1,
               biasv, startv, rsv, rpv,
               gsem0, gsem1, lsem0, lsem1, isem0, isem1):
    chunkb = (chunk0, chunk1)
    idxb = (idx0, idx1)
    valb = (val0, val1)
    rlocb = (rloc0, rloc1)
    gsemb = (gsem0, gsem1)
    lsemb = (lsem0, lsem1)
    isemb = (isem0, isem1)

    cid = lax.axis_index("c")
    sid = lax.axis_index("s")
    wid = cid * 16 + sid
    base = wid * RPW

    pltpu.sync_copy(starts, startv)
    se = startv[pl.ds(wid, 16)]
    lo = se[0]
    hi = se[1]

    pltpu.sync_copy(bias.at[pl.ds(base, RPW)], biasv)

    def init_rows(g, carry):
        bv16 = biasv[pl.ds(g * 16, 16)]
        for l in range(16):
            bv = jnp.full((16,), bv16[l], jnp.float32)
            for j in range(B // 16):
                accv[g * 16 + l, pl.ds(j * 16, 16)] = bv
        return carry

    lax.fori_loop(0, RPW // 16, init_rows, 0)

    for j in range(B // 16):
        rsv[pl.ds(j * 16, 16)] = jnp.zeros((16,), jnp.float32)
    rpv[pl.ds(0, 16)] = jnp.zeros((16,), jnp.int32)

    c0 = lo // K
    c1 = lax.div(hi + (K - 1), K)

    def issue_idx(b, ci):
        pltpu.async_copy(cols.at[pl.ds(ci * K, K)], idxb[b], isemb[b])

    def issue(b, ci):
        off = ci * K
        pltpu.make_async_copy(cols.at[pl.ds(0, K)], idxb[b], isemb[b]).wait()
        pltpu.async_copy(xT.at[idxb[b]], chunkb[b], gsemb[b])
        pltpu.async_copy(rows.at[pl.ds(off, K)], rlocb[b], lsemb[b])
        pltpu.async_copy(vals.at[pl.ds(off, K)], valb[b], lsemb[b])

    def drain(b):
        pltpu.make_async_copy(rows.at[pl.ds(0, K)], rlocb[b], lsemb[b]).wait()
        pltpu.make_async_copy(vals.at[pl.ds(0, K)], valb[b], lsemb[b]).wait()
        pltpu.make_async_copy(xT.at[idxb[b]], chunkb[b], gsemb[b]).wait()

    def compute(b, ci):
        off = ci * K
        drain(b)

        @pl.when(ci + 2 < c1)
        def _():
            issue_idx(b, ci + 2)

        for g in range(K // 16):
            gi = off + g * 16 + lax.iota(jnp.int32, 16)
            m = (gi >= lo) & (gi < hi)
            valb[b][pl.ds(g * 16, 16)] = jnp.where(
                m, valb[b][pl.ds(g * 16, 16)], jnp.float32(0.0))
            r = rlocb[b][pl.ds(g * 16, 16)] - base
            rlocb[b][pl.ds(g * 16, 16)] = jnp.clip(r, 0, RPW - 1)

        NJ = B // 16

        run0 = tuple(rsv[pl.ds(j * 16, 16)] for j in range(NJ))
        rp0 = rpv[pl.ds(0, 16)][0]

        def acc_body(g, carry):
            rp = carry[0]
            a = list(carry[1:])
            r16 = rlocb[b][pl.ds(g * 16, 16)]
            v16 = valb[b][pl.ds(g * 16, 16)]

            def stage(l):
                v = v16[l]
                e = g * 16 + l
                return [v * chunkb[b][e, pl.ds(j * 16, 16)]
                        for j in range(NJ)]

            prods = stage(0)
            for l in range(16):
                nxt = stage(l + 1) if l + 1 < 16 else None
                r = r16[l]
                ch = r != rp

                @pl.when(ch)
                def _(a=a, rp=rp):
                    for j in range(NJ):
                        plsc.addupdate(accv.at[rp, pl.ds(j * 16, 16)], a[j])

                keep = jnp.where(ch, jnp.float32(0.0), jnp.float32(1.0))
                a = [a[j] * keep + prods[j] for j in range(NJ)]
                rp = r
                prods = nxt
            return (rp, *a)

        fin = lax.fori_loop(0, K // 16, acc_body, (rp0, *run0))
        for j in range(NJ):
            rsv[pl.ds(j * 16, 16)] = fin[1 + j]
        rpv[pl.ds(0, 16)] = jnp.full((16,), fin[0], jnp.int32)

    @pl.when(c0 < c1)
    def _():
        issue_idx(0, c0)
        issue(0, c0)

    @pl.when(c0 + 1 < c1)
    def _():
        issue_idx(1, c0 + 1)
        issue(1, c0 + 1)

    npairs = lax.div(c1 - c0 + 1, 2)

    def pair_body(t, carry):
        for b in range(2):
            ci = c0 + t * 2 + b

            @pl.when(ci < c1)
            def _():
                compute(b, ci)

                @pl.when(ci + 2 < c1)
                def _():
                    issue(b, ci + 2)

        return carry

    lax.fori_loop(0, npairs, pair_body, 0)

    rp = rpv[pl.ds(0, 16)][0]
    for j in range(B // 16):
        plsc.addupdate(accv.at[rp, pl.ds(j * 16, 16)], rsv[pl.ds(j * 16, 16)])

    pltpu.sync_copy(accv, out.at[pl.ds(base, RPW)])


@jax.jit
def _spmm(xT, vals, rows, cols, bias, starts):
    mesh = plsc.VectorSubcoreMesh(core_axis_name="c", subcore_axis_name="s")
    run = functools.partial(
        pl.kernel,
        mesh=mesh,
        out_type=jax.ShapeDtypeStruct((OUT_F, B), jnp.float32),
        scratch_types=[
            pltpu.VMEM((RPW, B), jnp.float32),
            pltpu.VMEM((K, B), jnp.float32),
            pltpu.VMEM((K, B), jnp.float32),
            pltpu.VMEM((K,), jnp.int32),
            pltpu.VMEM((K,), jnp.int32),
            pltpu.VMEM((K,), jnp.float32),
            pltpu.VMEM((K,), jnp.float32),
            pltpu.VMEM((K,), jnp.int32),
            pltpu.VMEM((K,), jnp.int32),
            pltpu.VMEM((RPW,), jnp.float32),
            pltpu.VMEM((NST,), jnp.int32),
            pltpu.VMEM((B,), jnp.float32),
            pltpu.VMEM((16,), jnp.int32),
            pltpu.SemaphoreType.DMA,
            pltpu.SemaphoreType.DMA,
            pltpu.SemaphoreType.DMA,
            pltpu.SemaphoreType.DMA,
            pltpu.SemaphoreType.DMA,
            pltpu.SemaphoreType.DMA,
        ],
    )(_spmm_body)
    return run(xT, vals, rows, cols, bias, starts)


def kernel(_input, values, bias, rows, cols):
    rows32 = rows.astype(jnp.int32)
    cols32 = cols.astype(jnp.int32)
    nnz = rows32.shape[0]
    pad = (-nnz) % K
    rows_p = jnp.pad(rows32, (0, pad))
    cols_p = jnp.pad(cols32, (0, pad))
    vals_p = jnp.pad(values, (0, pad))
    bounds = jnp.arange(NW + 1, dtype=jnp.int32) * RPW
    starts = jnp.searchsorted(rows32, bounds).astype(jnp.int32)
    starts = jnp.pad(starts, (0, NST - (NW + 1)))
    xT = _input.T.copy()
    out_t = _spmm(xT, vals_p, rows_p, cols_p, bias, starts)
    return out_t.T
